# baseline (speedup 1.0000x reference)
import functools

import jax
import jax.numpy as jnp
from jax import lax
from jax.experimental import pallas as pl
from jax.experimental.pallas import tpu as pltpu
from jax.experimental.pallas import tpu_sc as plsc

F32 = jnp.float32
NC, NS, LANES = 2, 16, 16
CHUNK = 80


def _mesh():
    return plsc.VectorSubcoreMesh(
        core_axis_name="c", subcore_axis_name="s", num_cores=NC, num_subcores=NS
    )


_SC_PARAMS = pltpu.CompilerParams(needs_layout_passes=False)


def _dot3(a, b):
    ah = a.astype(jnp.bfloat16)
    al = (a - ah.astype(F32)).astype(jnp.bfloat16)
    bh = b.astype(jnp.bfloat16)
    bl = (b - bh.astype(F32)).astype(jnp.bfloat16)

    def d(p, q):
        return jnp.dot(p, q, preferred_element_type=F32)

    return d(ah, bh) + d(ah, bl) + d(al, bh)


def _sc_degree(dst, n):
    e = dst.shape[0]
    nw = NC * NS
    epw = e // nw
    full = epw // LANES
    rem = epw - full * LANES
    pad = (LANES - rem) % LANES

    @functools.partial(
        pl.kernel,
        out_type=jax.ShapeDtypeStruct((nw * n,), F32),
        mesh=_mesh(),
        compiler_params=_SC_PARAMS,
        scratch_types=[
            pltpu.VMEM((epw + pad,), jnp.int32),
            pltpu.VMEM((n,), F32),
        ],
    )
    def k(dst_hbm, out_hbm, idx_v, hist_v):
        cid = lax.axis_index("c")
        sid = lax.axis_index("s")
        w = sid * NC + cid
        zv = jnp.zeros((LANES,), F32)

        def zb(i, carry):
            hist_v[pl.ds(i * LANES, LANES)] = zv
            return carry

        lax.fori_loop(0, n // LANES, zb, 0)
        pltpu.sync_copy(dst_hbm.at[pl.ds(w * epw, epw)], idx_v.at[pl.ds(0, epw)])
        ones = jnp.ones((LANES,), F32)

        def body(i, carry):
            idx = idx_v[pl.ds(i * LANES, LANES)]
            plsc.addupdate_scatter(hist_v, [idx], ones)
            return carry

        lax.fori_loop(0, full, body, 0)
        if rem:
            idx = idx_v[pl.ds(full * LANES, LANES)]
            mask = lax.iota(jnp.int32, LANES) < rem
            idx = jnp.where(mask, idx, 0)
            plsc.addupdate_scatter(hist_v, [idx], ones, mask=mask)
        pltpu.sync_copy(hist_v, out_hbm.at[pl.ds(w * n, n)])

    return k(dst)


def _tc_dis(parts_t):
    n, nw = parts_t.shape
    bn = 400

    def body(p_ref, o_ref):
        s = jnp.sum(p_ref[...], axis=1, keepdims=True) + 1.0
        o_ref[...] = lax.rsqrt(s)

    return pl.pallas_call(
        body,
        grid=(n // bn,),
        in_specs=[pl.BlockSpec((bn, nw), lambda i: (i, 0))],
        out_specs=pl.BlockSpec((bn, 1), lambda i: (i, 0)),
        out_shape=jax.ShapeDtypeStruct((n, 1), F32),
    )(parts_t)


def _tc_mm_scale_split(xin, w, dis):
    n, f = xin.shape
    hh = w.shape[1] // 2
    bn = 1000
    nb = n // bn

    def body(x_ref, w_ref, d_ref, o_ref):
        o_ref[...] = _dot3(x_ref[...], w_ref[...]) * d_ref[...]

    return pl.pallas_call(
        body,
        grid=(2, nb),
        in_specs=[
            pl.BlockSpec((bn, f), lambda c, i: (i, 0)),
            pl.BlockSpec((f, hh), lambda c, i: (0, c)),
            pl.BlockSpec((bn, 1), lambda c, i: (i, 0)),
        ],
        out_specs=pl.BlockSpec((bn, hh), lambda c, i: (c * nb + i, 0)),
        out_shape=jax.ShapeDtypeStruct((2 * n, hh), F32),
    )(xin, w, dis)


def _sc_scatter(ts_flat, src, dst, zeros, n):
    e = src.shape[0]
    hh = ts_flat.shape[1]
    ept = e // NS
    nck = ept // CHUNK
    rpt = 640
    rlast = n - (NS - 1) * rpt

    @functools.partial(
        pl.kernel,
        out_type=jax.ShapeDtypeStruct((NC * n, hh), F32),
        mesh=_mesh(),
        compiler_params=_SC_PARAMS,
        scratch_types=[
            pltpu.VMEM((ept,), jnp.int32),
            pltpu.VMEM((CHUNK,), jnp.int32),
            pltpu.VMEM((CHUNK,), jnp.int32),
            pltpu.VMEM((CHUNK, hh), F32),
            pltpu.VMEM((CHUNK, hh), F32),
            pltpu.VMEM_SHARED((n, hh), F32),
            pltpu.SemaphoreType.DMA,
            pltpu.SemaphoreType.DMA,
        ],
    )
    def k(ts_hbm, src_hbm, dst_hbm, z_hbm, out_hbm,
          idxs_all, idxd0, idxd1, rows0, rows1, acc_sh, sem0, sem1):
        cid = lax.axis_index("c")
        sid = lax.axis_index("s")

        @pl.when(sid < NS - 1)
        def _():
            pltpu.sync_copy(z_hbm, acc_sh.at[pl.ds(sid * rpt, rpt)])

        @pl.when(sid == NS - 1)
        def _():
            pltpu.sync_copy(
                z_hbm.at[pl.ds(0, rlast)],
                acc_sh.at[pl.ds((NS - 1) * rpt, rlast)],
            )

        base = sid * ept
        pltpu.sync_copy(src_hbm.at[pl.ds(base, ept)], idxs_all)
        offv = jnp.full((LANES,), cid * n, jnp.int32)

        def addoff(i, c2):
            sl = pl.ds(i * LANES, LANES)
            idxs_all[sl] = idxs_all[sl] + offv
            return c2

        lax.fori_loop(0, ept // LANES, addoff, 0)
        plsc.subcore_barrier()

        def issue(kk, idxd, rows, sem):
            pltpu.sync_copy(dst_hbm.at[pl.ds(base + kk * CHUNK, CHUNK)], idxd)
            pltpu.async_copy(
                ts_hbm.at[idxs_all.at[pl.ds(kk * CHUNK, CHUNK)]], rows, sem
            )

        def drain(kk, rows, sem):
            pltpu.make_async_copy(
                ts_hbm.at[idxs_all.at[pl.ds(kk * CHUNK, CHUNK)]], rows, sem
            ).wait()

        def commit(rows, idxd):
            pltpu.sync_copy(rows, acc_sh.at[idxd], add=True)

        issue(0, idxd0, rows0, sem0)

        def pair(i, carry):
            k0 = 2 * i
            issue(k0 + 1, idxd1, rows1, sem1)
            drain(k0, rows0, sem0)
            commit(rows0, idxd0)
            issue(k0 + 2, idxd0, rows0, sem0)
            drain(k0 + 1, rows1, sem1)
            commit(rows1, idxd1)
            return carry

        lax.fori_loop(0, (nck - 1) // 2, pair, 0)
        drain(nck - 1, rows0, sem0)
        commit(rows0, idxd0)
        plsc.subcore_barrier()

        @pl.when(sid < NS - 1)
        def _():
            pltpu.sync_copy(
                acc_sh.at[pl.ds(sid * rpt, rpt)],
                out_hbm.at[pl.ds(cid * n + sid * rpt, rpt)],
            )

        @pl.when(sid == NS - 1)
        def _():
            pltpu.sync_copy(
                acc_sh.at[pl.ds((NS - 1) * rpt, rlast)],
                out_hbm.at[pl.ds(cid * n + (NS - 1) * rpt, rlast)],
            )

    return k(ts_flat, src, dst, zeros)


def _tc_epilogue(acc, ts, dis, b):
    hh = ts.shape[1]
    n = ts.shape[0] // 2
    bn = 1000
    nb = n // bn

    def body(a_ref, t_ref, d_ref, b_ref, o_ref):
        o_ref[...] = jnp.maximum((a_ref[...] + t_ref[...]) * d_ref[...] + b_ref[0], 0.0)

    return pl.pallas_call(
        body,
        grid=(2, nb),
        in_specs=[
            pl.BlockSpec((bn, hh), lambda c, i: (c * nb + i, 0)),
            pl.BlockSpec((bn, hh), lambda c, i: (c * nb + i, 0)),
            pl.BlockSpec((bn, 1), lambda c, i: (i, 0)),
            pl.BlockSpec((1, 1, hh), lambda c, i: (c, 0, 0)),
        ],
        out_specs=pl.BlockSpec((bn, hh), lambda c, i: (i, c)),
        out_shape=jax.ShapeDtypeStruct((n, 2 * hh), F32),
    )(acc, ts, dis, b)


def _tc_mm_ab(h2, wab):
    n, f = h2.shape
    hh = wab.shape[1] // 2
    bn = 1000
    nb = n // bn

    def body(x_ref, w_ref, o_ref):
        o_ref[...] = _dot3(x_ref[...], w_ref[...])

    return pl.pallas_call(
        body,
        grid=(2, 2, nb),
        in_specs=[
            pl.BlockSpec((bn, f), lambda s, c, i: (i, 0)),
            pl.BlockSpec((f, hh), lambda s, c, i: (s, c)),
        ],
        out_specs=pl.BlockSpec((bn, hh), lambda s, c, i: ((s * 2 + c) * nb + i, 0)),
        out_shape=jax.ShapeDtypeStruct((4 * n, hh), F32),
    )(h2, wab)


def _sc_gather_add(tab, src, dst, n):
    e = src.shape[0]
    hh = tab.shape[1]
    ept = e // NS
    ck = next(c for c in (80, 40, 16, 8)
              if ept % c == 0 and (ept // c) % 2 == 1)
    nck = ept // ck

    @functools.partial(
        pl.kernel,
        out_type=jax.ShapeDtypeStruct((NC * e, hh), F32),
        mesh=_mesh(),
        compiler_params=_SC_PARAMS,
        scratch_types=[
            pltpu.VMEM((ept,), jnp.int32),
            pltpu.VMEM((ept,), jnp.int32),
            pltpu.VMEM((ck, hh), F32),
            pltpu.VMEM((ck, hh), F32),
            pltpu.VMEM((ck, hh), F32),
            pltpu.VMEM((ck, hh), F32),
            pltpu.SemaphoreType.DMA,
            pltpu.SemaphoreType.DMA,
            pltpu.SemaphoreType.DMA,
            pltpu.SemaphoreType.DMA,
        ],
    )
    def k(tab_hbm, src_hbm, dst_hbm, out_hbm,
          idxs_all, idxd_all, a0, b0, a1, b1, sa0, sb0, sa1, sb1):
        cid = lax.axis_index("c")
        sid = lax.axis_index("s")
        base = sid * ept
        pltpu.sync_copy(src_hbm.at[pl.ds(base, ept)], idxs_all)
        pltpu.sync_copy(dst_hbm.at[pl.ds(base, ept)], idxd_all)
        offa = jnp.full((LANES,), cid * n, jnp.int32)
        offb = jnp.full((LANES,), (2 + cid) * n, jnp.int32)

        def addoff(i, c2):
            sl = pl.ds(i * LANES, LANES)
            idxs_all[sl] = idxs_all[sl] + offa
            idxd_all[sl] = idxd_all[sl] + offb
            return c2

        lax.fori_loop(0, ept // LANES, addoff, 0)

        def issue(kk, ba, bb, sa, sb):
            sl = pl.ds(kk * ck, ck)
            pltpu.async_copy(tab_hbm.at[idxs_all.at[sl]], ba, sa)
            pltpu.async_copy(tab_hbm.at[idxd_all.at[sl]], bb, sb)

        def finish(kk, ba, bb, sa, sb):
            sl = pl.ds(kk * ck, ck)
            pltpu.make_async_copy(tab_hbm.at[idxs_all.at[sl]], ba, sa).wait()
            pltpu.make_async_copy(tab_hbm.at[idxd_all.at[sl]], bb, sb).wait()

            def addrow(r, c2):
                for j in range(hh // LANES):
                    s2 = pl.ds(j * LANES, LANES)
                    ba[r, s2] = ba[r, s2] + bb[r, s2]
                return c2

            lax.fori_loop(0, ck, addrow, 0)
            pltpu.sync_copy(ba, out_hbm.at[pl.ds(cid * e + base + kk * ck, ck)])

        issue(0, a0, b0, sa0, sb0)

        def pair(i, carry):
            k0 = 2 * i
            issue(k0 + 1, a1, b1, sa1, sb1)
            finish(k0, a0, b0, sa0, sb0)
            issue(k0 + 2, a0, b0, sa0, sb0)
            finish(k0 + 1, a1, b1, sa1, sb1)
            return carry

        lax.fori_loop(0, (nck - 1) // 2, pair, 0)
        finish(nck - 1, a0, b0, sa0, sb0)

    return k(tab, src, dst)


def _tc_edge(g, ea, w_ea, bc1, wc2_bf16, bc2, wc3_row, bc3):
    hh = g.shape[1]
    e = g.shape[0] // 2
    be = next(b for b in (6400, 3200, 1600, 640, 128) if e % b == 0)
    nb = e // be

    def body(g0_ref, g1_ref, eat_ref, wea_ref, b1_ref, w2_ref, b2_ref, w3_ref,
             b3_ref, o_ref):
        u = jnp.concatenate([g0_ref[...], g1_ref[...]], axis=1)
        ea = jnp.transpose(eat_ref[...])
        u = u + ea[:, 0:1] * wea_ref[0:1, :] + ea[:, 1:2] * wea_ref[1:2, :]
        z = jnp.maximum(u + b1_ref[...], 0.0).astype(jnp.bfloat16)
        z = jnp.dot(z, w2_ref[...], preferred_element_type=F32) + b2_ref[...]
        z = jnp.maximum(z, 0.0)
        z = jnp.sum(z * w3_ref[...], axis=1, keepdims=True) + b3_ref[...]
        o_ref[...] = jnp.transpose(jax.nn.sigmoid(z))

    return pl.pallas_call(
        body,
        grid=(nb,),
        in_specs=[
            pl.BlockSpec((be, hh), lambda i: (i, 0)),
            pl.BlockSpec((be, hh), lambda i: (nb + i, 0)),
            pl.BlockSpec((2, be), lambda i: (0, i)),
            pl.BlockSpec((2, 2 * hh), lambda i: (0, 0)),
            pl.BlockSpec((1, 2 * hh), lambda i: (0, 0)),
            pl.BlockSpec((2 * hh, 32), lambda i: (0, 0)),
            pl.BlockSpec((1, 32), lambda i: (0, 0)),
            pl.BlockSpec((1, 32), lambda i: (0, 0)),
            pl.BlockSpec((1, 1), lambda i: (0, 0)),
        ],
        out_specs=pl.BlockSpec((1, be), lambda i: (0, i)),
        out_shape=jax.ShapeDtypeStruct((1, e), F32),
    )(g, g, ea, w_ea, bc1, wc2_bf16, bc2, wc3_row, bc3)


def kernel(x, edge_attr, W1, b1, W2, b2, Wc1, bc1, Wc2, bc2, Wc3, bc3, edge_index):
    n, _ = x.shape
    e = edge_index.shape[1]
    h = W1.shape[1]
    hh = h // 2
    src = edge_index[0]
    dst = edge_index[1]

    parts = _sc_degree(dst, n)
    dis = _tc_dis(parts.reshape(NC * NS, n).T)
    zeros = jnp.zeros((640, hh), F32)

    ts1 = _tc_mm_scale_split(x, W1, dis)
    acc1 = _sc_scatter(ts1, src, dst, zeros, n)
    h1 = _tc_epilogue(acc1, ts1, dis, b1.reshape(2, 1, hh))

    ts2 = _tc_mm_scale_split(h1, W2, dis)
    acc2 = _sc_scatter(ts2, src, dst, zeros, n)
    h2 = _tc_epilogue(acc2, ts2, dis, b2.reshape(2, 1, hh))

    tab = _tc_mm_ab(h2, Wc1[: 2 * h])
    eh = e // 2
    eat = edge_attr.T[0:2]
    mlp_w = (Wc1[2 * h :], bc1.reshape(1, h), Wc2.astype(jnp.bfloat16),
             bc2.reshape(1, -1), Wc3.reshape(1, 32), bc3.reshape(1, 1))
    ga = _sc_gather_add(tab, src[:eh], dst[:eh], n)
    gb = _sc_gather_add(tab, src[eh:], dst[eh:], n)
    za = _tc_edge(ga, eat[:, :eh], *mlp_w)
    zb = _tc_edge(gb, eat[:, eh:], *mlp_w)
    return jnp.concatenate([za, zb], axis=1).T


# speedup vs baseline: 1.0825x; 1.0825x over previous
import functools

import jax
import jax.numpy as jnp
from jax import lax
from jax.experimental import pallas as pl
from jax.experimental.pallas import tpu as pltpu
from jax.experimental.pallas import tpu_sc as plsc

F32 = jnp.float32
NC, NS, LANES = 2, 16, 16
CHUNK = 80


def _mesh():
    return plsc.VectorSubcoreMesh(
        core_axis_name="c", subcore_axis_name="s", num_cores=NC, num_subcores=NS
    )


_SC_PARAMS = pltpu.CompilerParams(needs_layout_passes=False)


def _dot3(a, b):
    ah = a.astype(jnp.bfloat16)
    al = (a - ah.astype(F32)).astype(jnp.bfloat16)
    bh = b.astype(jnp.bfloat16)
    bl = (b - bh.astype(F32)).astype(jnp.bfloat16)

    def d(p, q):
        return jnp.dot(p, q, preferred_element_type=F32)

    return d(ah, bh) + d(ah, bl) + d(al, bh)


def _sc_degree(dst, n):
    e = dst.shape[0]
    nw = NC * NS
    epw = e // nw
    full = epw // LANES
    rem = epw - full * LANES
    pad = (LANES - rem) % LANES

    @functools.partial(
        pl.kernel,
        out_type=jax.ShapeDtypeStruct((nw * n,), F32),
        mesh=_mesh(),
        compiler_params=_SC_PARAMS,
        scratch_types=[
            pltpu.VMEM((epw + pad,), jnp.int32),
            pltpu.VMEM((n,), F32),
        ],
    )
    def k(dst_hbm, out_hbm, idx_v, hist_v):
        cid = lax.axis_index("c")
        sid = lax.axis_index("s")
        w = sid * NC + cid
        zv = jnp.zeros((LANES,), F32)

        def zb(i, carry):
            hist_v[pl.ds(i * LANES, LANES)] = zv
            return carry

        lax.fori_loop(0, n // LANES, zb, 0)
        pltpu.sync_copy(dst_hbm.at[pl.ds(w * epw, epw)], idx_v.at[pl.ds(0, epw)])
        ones = jnp.ones((LANES,), F32)

        def body(i, carry):
            idx = idx_v[pl.ds(i * LANES, LANES)]
            plsc.addupdate_scatter(hist_v, [idx], ones)
            return carry

        lax.fori_loop(0, full, body, 0)
        if rem:
            idx = idx_v[pl.ds(full * LANES, LANES)]
            mask = lax.iota(jnp.int32, LANES) < rem
            idx = jnp.where(mask, idx, 0)
            plsc.addupdate_scatter(hist_v, [idx], ones, mask=mask)
        pltpu.sync_copy(hist_v, out_hbm.at[pl.ds(w * n, n)])

    return k(dst)


def _tc_dis(parts_t):
    n, nw = parts_t.shape
    bn = 400

    def body(p_ref, o_ref):
        s = jnp.sum(p_ref[...], axis=1, keepdims=True) + 1.0
        o_ref[...] = lax.rsqrt(s)

    return pl.pallas_call(
        body,
        grid=(n // bn,),
        in_specs=[pl.BlockSpec((bn, nw), lambda i: (i, 0))],
        out_specs=pl.BlockSpec((bn, 1), lambda i: (i, 0)),
        out_shape=jax.ShapeDtypeStruct((n, 1), F32),
    )(parts_t)


def _tc_mm_scale_split(xin, w, dis):
    n, f = xin.shape
    hh = w.shape[1] // 2
    bn = 1000
    nb = n // bn

    def body(x_ref, w_ref, d_ref, o_ref):
        o_ref[...] = _dot3(x_ref[...], w_ref[...]) * d_ref[...]

    return pl.pallas_call(
        body,
        grid=(2, nb),
        in_specs=[
            pl.BlockSpec((bn, f), lambda c, i: (i, 0)),
            pl.BlockSpec((f, hh), lambda c, i: (0, c)),
            pl.BlockSpec((bn, 1), lambda c, i: (i, 0)),
        ],
        out_specs=pl.BlockSpec((bn, hh), lambda c, i: (c * nb + i, 0)),
        out_shape=jax.ShapeDtypeStruct((2 * n, hh), F32),
    )(xin, w, dis)


def _sc_scatter(ts_flat, src, dst, zeros, n):
    e = src.shape[0]
    hh = ts_flat.shape[1]
    ept = e // NS
    nck = ept // CHUNK
    rpt = 640
    rlast = n - (NS - 1) * rpt

    @functools.partial(
        pl.kernel,
        out_type=jax.ShapeDtypeStruct((NC * n, hh), F32),
        mesh=_mesh(),
        compiler_params=_SC_PARAMS,
        scratch_types=[
            pltpu.VMEM((ept,), jnp.int32),
            pltpu.VMEM((nck, CHUNK), jnp.int32),
            pltpu.VMEM((CHUNK, hh), F32),
            pltpu.VMEM((CHUNK, hh), F32),
            pltpu.VMEM_SHARED((n, hh), F32),
            pltpu.SemaphoreType.DMA,
            pltpu.SemaphoreType.DMA,
        ],
    )
    def k(ts_hbm, src_hbm, dst3_hbm, z_hbm, out_hbm,
          idxs_all, idxd2, rows0, rows1, acc_sh, sem0, sem1):
        cid = lax.axis_index("c")
        sid = lax.axis_index("s")

        @pl.when(sid < NS - 1)
        def _():
            pltpu.sync_copy(z_hbm, acc_sh.at[pl.ds(sid * rpt, rpt)])

        @pl.when(sid == NS - 1)
        def _():
            pltpu.sync_copy(
                z_hbm.at[pl.ds(0, rlast)],
                acc_sh.at[pl.ds((NS - 1) * rpt, rlast)],
            )

        base = sid * ept
        pltpu.sync_copy(src_hbm.at[pl.ds(base, ept)], idxs_all)
        pltpu.sync_copy(dst3_hbm.at[sid], idxd2)
        offv = jnp.full((LANES,), cid * n, jnp.int32)

        def addoff(i, c2):
            sl = pl.ds(i * LANES, LANES)
            idxs_all[sl] = idxs_all[sl] + offv
            return c2

        lax.fori_loop(0, ept // LANES, addoff, 0)
        plsc.subcore_barrier()

        def issue(kk, rows, sem):
            pltpu.async_copy(
                ts_hbm.at[idxs_all.at[pl.ds(kk * CHUNK, CHUNK)]], rows, sem
            )

        def drain(kk, rows, sem):
            pltpu.make_async_copy(
                ts_hbm.at[idxs_all.at[pl.ds(kk * CHUNK, CHUNK)]], rows, sem
            ).wait()

        def commit(kk, rows):
            pltpu.sync_copy(rows, acc_sh.at[idxd2.at[kk]], add=True)

        issue(0, rows0, sem0)

        def pair(i, carry):
            k0 = 2 * i
            issue(k0 + 1, rows1, sem1)
            drain(k0, rows0, sem0)
            commit(k0, rows0)
            issue(k0 + 2, rows0, sem0)
            drain(k0 + 1, rows1, sem1)
            commit(k0 + 1, rows1)
            return carry

        lax.fori_loop(0, (nck - 1) // 2, pair, 0)
        drain(nck - 1, rows0, sem0)
        commit(nck - 1, rows0)
        plsc.subcore_barrier()

        @pl.when(sid < NS - 1)
        def _():
            pltpu.sync_copy(
                acc_sh.at[pl.ds(sid * rpt, rpt)],
                out_hbm.at[pl.ds(cid * n + sid * rpt, rpt)],
            )

        @pl.when(sid == NS - 1)
        def _():
            pltpu.sync_copy(
                acc_sh.at[pl.ds((NS - 1) * rpt, rlast)],
                out_hbm.at[pl.ds(cid * n + (NS - 1) * rpt, rlast)],
            )

    return k(ts_flat, src, dst.reshape(NS, nck, CHUNK), zeros)


def _tc_epilogue(acc, ts, dis, b):
    hh = ts.shape[1]
    n = ts.shape[0] // 2
    bn = 1000
    nb = n // bn

    def body(a_ref, t_ref, d_ref, b_ref, o_ref):
        o_ref[...] = jnp.maximum((a_ref[...] + t_ref[...]) * d_ref[...] + b_ref[0], 0.0)

    return pl.pallas_call(
        body,
        grid=(2, nb),
        in_specs=[
            pl.BlockSpec((bn, hh), lambda c, i: (c * nb + i, 0)),
            pl.BlockSpec((bn, hh), lambda c, i: (c * nb + i, 0)),
            pl.BlockSpec((bn, 1), lambda c, i: (i, 0)),
            pl.BlockSpec((1, 1, hh), lambda c, i: (c, 0, 0)),
        ],
        out_specs=pl.BlockSpec((bn, hh), lambda c, i: (i, c)),
        out_shape=jax.ShapeDtypeStruct((n, 2 * hh), F32),
    )(acc, ts, dis, b)


def _tc_mm_ab(h2, wab):
    n, f = h2.shape
    hh = wab.shape[1] // 2
    bn = 1000
    nb = n // bn

    def body(x_ref, w_ref, o_ref):
        o_ref[...] = _dot3(x_ref[...], w_ref[...])

    return pl.pallas_call(
        body,
        grid=(2, 2, nb),
        in_specs=[
            pl.BlockSpec((bn, f), lambda s, c, i: (i, 0)),
            pl.BlockSpec((f, hh), lambda s, c, i: (s, c)),
        ],
        out_specs=pl.BlockSpec((bn, hh), lambda s, c, i: ((s * 2 + c) * nb + i, 0)),
        out_shape=jax.ShapeDtypeStruct((4 * n, hh), F32),
    )(h2, wab)


def _sc_gather_add(tab, src, dst, n):
    e = src.shape[0]
    hh = tab.shape[1]
    ept = e // NS
    ck = next(c for c in (80, 40, 16, 8)
              if ept % c == 0 and (ept // c) % 2 == 1)
    nck = ept // ck

    @functools.partial(
        pl.kernel,
        out_type=jax.ShapeDtypeStruct((NC * e, hh), F32),
        mesh=_mesh(),
        compiler_params=_SC_PARAMS,
        scratch_types=[
            pltpu.VMEM((ept,), jnp.int32),
            pltpu.VMEM((ept,), jnp.int32),
            pltpu.VMEM((ck, hh), F32),
            pltpu.VMEM((ck, hh), F32),
            pltpu.VMEM((ck, hh), F32),
            pltpu.VMEM((ck, hh), F32),
            pltpu.SemaphoreType.DMA,
            pltpu.SemaphoreType.DMA,
            pltpu.SemaphoreType.DMA,
            pltpu.SemaphoreType.DMA,
        ],
    )
    def k(tab_hbm, src_hbm, dst_hbm, out_hbm,
          idxs_all, idxd_all, a0, b0, a1, b1, sa0, sb0, sa1, sb1):
        cid = lax.axis_index("c")
        sid = lax.axis_index("s")
        base = sid * ept
        pltpu.sync_copy(src_hbm.at[pl.ds(base, ept)], idxs_all)
        pltpu.sync_copy(dst_hbm.at[pl.ds(base, ept)], idxd_all)
        offa = jnp.full((LANES,), cid * n, jnp.int32)
        offb = jnp.full((LANES,), (2 + cid) * n, jnp.int32)

        def addoff(i, c2):
            sl = pl.ds(i * LANES, LANES)
            idxs_all[sl] = idxs_all[sl] + offa
            idxd_all[sl] = idxd_all[sl] + offb
            return c2

        lax.fori_loop(0, ept // LANES, addoff, 0)

        def issue(kk, ba, bb, sa, sb):
            sl = pl.ds(kk * ck, ck)
            pltpu.async_copy(tab_hbm.at[idxs_all.at[sl]], ba, sa)
            pltpu.async_copy(tab_hbm.at[idxd_all.at[sl]], bb, sb)

        def finish(kk, ba, bb, sa, sb):
            sl = pl.ds(kk * ck, ck)
            pltpu.make_async_copy(tab_hbm.at[idxs_all.at[sl]], ba, sa).wait()
            pltpu.make_async_copy(tab_hbm.at[idxd_all.at[sl]], bb, sb).wait()

            def addrow(r, c2):
                for j in range(hh // LANES):
                    s2 = pl.ds(j * LANES, LANES)
                    ba[r, s2] = ba[r, s2] + bb[r, s2]
                return c2

            lax.fori_loop(0, ck, addrow, 0)
            pltpu.sync_copy(ba, out_hbm.at[pl.ds(cid * e + base + kk * ck, ck)])

        issue(0, a0, b0, sa0, sb0)

        def pair(i, carry):
            k0 = 2 * i
            issue(k0 + 1, a1, b1, sa1, sb1)
            finish(k0, a0, b0, sa0, sb0)
            issue(k0 + 2, a0, b0, sa0, sb0)
            finish(k0 + 1, a1, b1, sa1, sb1)
            return carry

        lax.fori_loop(0, (nck - 1) // 2, pair, 0)
        finish(nck - 1, a0, b0, sa0, sb0)

    return k(tab, src, dst)


def _tc_edge(g, ea, w_ea, bc1, wc2_bf16, bc2, wc3_row, bc3):
    hh = g.shape[1]
    e = g.shape[0] // 2
    be = next(b for b in (6400, 3200, 1600, 640, 128) if e % b == 0)
    nb = e // be

    def body(g0_ref, g1_ref, eat_ref, wea_ref, b1_ref, w2_ref, b2_ref, w3_ref,
             b3_ref, o_ref):
        u = jnp.concatenate([g0_ref[...], g1_ref[...]], axis=1)
        ea = jnp.transpose(eat_ref[...])
        u = u + ea[:, 0:1] * wea_ref[0:1, :] + ea[:, 1:2] * wea_ref[1:2, :]
        z = jnp.maximum(u + b1_ref[...], 0.0).astype(jnp.bfloat16)
        z = jnp.dot(z, w2_ref[...], preferred_element_type=F32) + b2_ref[...]
        z = jnp.maximum(z, 0.0)
        z = jnp.sum(z * w3_ref[...], axis=1, keepdims=True) + b3_ref[...]
        o_ref[...] = jnp.transpose(jax.nn.sigmoid(z))

    return pl.pallas_call(
        body,
        grid=(nb,),
        in_specs=[
            pl.BlockSpec((be, hh), lambda i: (i, 0)),
            pl.BlockSpec((be, hh), lambda i: (nb + i, 0)),
            pl.BlockSpec((2, be), lambda i: (0, i)),
            pl.BlockSpec((2, 2 * hh), lambda i: (0, 0)),
            pl.BlockSpec((1, 2 * hh), lambda i: (0, 0)),
            pl.BlockSpec((2 * hh, 32), lambda i: (0, 0)),
            pl.BlockSpec((1, 32), lambda i: (0, 0)),
            pl.BlockSpec((1, 32), lambda i: (0, 0)),
            pl.BlockSpec((1, 1), lambda i: (0, 0)),
        ],
        out_specs=pl.BlockSpec((1, be), lambda i: (0, i)),
        out_shape=jax.ShapeDtypeStruct((1, e), F32),
    )(g, g, ea, w_ea, bc1, wc2_bf16, bc2, wc3_row, bc3)


def kernel(x, edge_attr, W1, b1, W2, b2, Wc1, bc1, Wc2, bc2, Wc3, bc3, edge_index):
    n, _ = x.shape
    e = edge_index.shape[1]
    h = W1.shape[1]
    hh = h // 2
    src = edge_index[0]
    dst = edge_index[1]

    parts = _sc_degree(dst, n)
    dis = _tc_dis(parts.reshape(NC * NS, n).T)
    zeros = jnp.zeros((640, hh), F32)

    ts1 = _tc_mm_scale_split(x, W1, dis)
    acc1 = _sc_scatter(ts1, src, dst, zeros, n)
    h1 = _tc_epilogue(acc1, ts1, dis, b1.reshape(2, 1, hh))

    ts2 = _tc_mm_scale_split(h1, W2, dis)
    acc2 = _sc_scatter(ts2, src, dst, zeros, n)
    h2 = _tc_epilogue(acc2, ts2, dis, b2.reshape(2, 1, hh))

    tab = _tc_mm_ab(h2, Wc1[: 2 * h])
    g = _sc_gather_add(tab, src, dst, n)
    zrow = _tc_edge(
        g, edge_attr.T[0:2], Wc1[2 * h :], bc1.reshape(1, h),
        Wc2.astype(jnp.bfloat16), bc2.reshape(1, -1), Wc3.reshape(1, 32),
        bc3.reshape(1, 1),
    )
    return zrow.T
